# Initial kernel scaffold; baseline (speedup 1.0000x reference)
import jax
import jax.numpy as jnp
from jax.experimental import pallas as pl


def kernel(pos_u, pos_v, neg_u, neg_v, u_table, v_table):
    raise NotImplementedError("write your pallas kernel here")


# speedup vs baseline: 3.0006x; 3.0006x over previous
import functools

import jax
import jax.numpy as jnp
from jax import lax
from jax.experimental import pallas as pl
from jax.experimental.pallas import tpu as pltpu
from jax.experimental.pallas import tpu_sc as plsc

NC = 2
NS = 16
LANES = 16
NW = NC * NS

CB = 16
GATHER_ROWS = 80


def _make_sc_scores(n_items, ctx, d, ipw):
    t_chunks = ipw // CB
    mesh = plsc.VectorSubcoreMesh(core_axis_name="c", subcore_axis_name="s")

    @functools.partial(
        pl.kernel,
        mesh=mesh,
        compiler_params=pltpu.CompilerParams(needs_layout_passes=False),
        out_type=jax.ShapeDtypeStruct((n_items,), jnp.float32),
        scratch_types=[
            pltpu.VMEM((ipw * ctx,), jnp.int32),
            pltpu.VMEM((ipw,), jnp.int32),
            pltpu.VMEM((GATHER_ROWS, d), jnp.float32),
            pltpu.VMEM((GATHER_ROWS, d), jnp.float32),
            pltpu.VMEM((GATHER_ROWS, d), jnp.float32),
            pltpu.VMEM((GATHER_ROWS, d), jnp.float32),
            pltpu.VMEM((GATHER_ROWS, d), jnp.float32),
            pltpu.VMEM((GATHER_ROWS, d), jnp.float32),
            pltpu.VMEM((GATHER_ROWS, d), jnp.float32),
            pltpu.VMEM((GATHER_ROWS, d), jnp.float32),
            pltpu.VMEM((CB, d), jnp.float32),
            pltpu.VMEM((CB, d), jnp.float32),
            pltpu.VMEM((ipw,), jnp.float32),
            pltpu.VMEM((CB, LANES), jnp.float32),
            pltpu.SemaphoreType.DMA,
            pltpu.SemaphoreType.DMA,
        ],
    )
    def sc_scores(uidx_hbm, vidx_hbm, ut_hbm, vt_hbm, out_hbm,
                  uidx_v, vidx_v,
                  r00, r01, r02, r03, r10, r11, r12, r13,
                  vr0, vr1, scores_v, pmat, sem0, sem1):
        wid = lax.axis_index("s") * NC + lax.axis_index("c")
        rows_sets = ((r00, r01, r02, r03), (r10, r11, r12, r13))
        vr_sets = (vr0, vr1)
        sems = (sem0, sem1)

        pltpu.sync_copy(uidx_hbm.at[pl.ds(wid * (ipw * ctx), ipw * ctx)], uidx_v)
        pltpu.sync_copy(vidx_hbm.at[pl.ds(wid * ipw, ipw)], vidx_v)

        def fire(t, s):
            rows, vr, sem = rows_sets[s], vr_sets[s], sems[s]
            bu = t * (CB * ctx)
            for g in range(4):
                pltpu.make_async_copy(
                    ut_hbm.at[uidx_v.at[pl.ds(bu + g * GATHER_ROWS, GATHER_ROWS)]],
                    rows[g], sem).start()
            pltpu.make_async_copy(
                vt_hbm.at[vidx_v.at[pl.ds(t * CB, CB)]], vr, sem).start()

        def drain(t, s):
            rows, vr, sem = rows_sets[s], vr_sets[s], sems[s]
            bu = t * (CB * ctx)
            for g in range(4):
                pltpu.make_async_copy(
                    ut_hbm.at[uidx_v.at[pl.ds(bu + g * GATHER_ROWS, GATHER_ROWS)]],
                    rows[g], sem).wait()
            pltpu.make_async_copy(
                vt_hbm.at[vidx_v.at[pl.ds(t * CB, CB)]], vr, sem).wait()

        nj = d // LANES
        lanes = lax.iota(jnp.int32, LANES)

        def compute(t, s):
            rows_set, vr = rows_sets[s], vr_sets[s]
            items_per_buf = GATHER_ROWS // ctx
            for sub in range(4):
                rows = rows_set[sub]

                def item_body(i, carry, _rows=rows, _sub=sub):
                    lane = _sub * items_per_buf + i
                    r0 = i * ctx
                    a = [_rows[r0, pl.ds(LANES * j, LANES)] for j in range(nj)]
                    for c in range(1, ctx):
                        for j in range(nj):
                            a[j] = a[j] + _rows[r0 + c, pl.ds(LANES * j, LANES)]
                    p = a[0] * vr[lane, pl.ds(0, LANES)]
                    for j in range(1, nj):
                        p = p + a[j] * vr[lane, pl.ds(LANES * j, LANES)]
                    pmat[lane, :] = p
                    return carry

                lax.fori_loop(0, items_per_buf, item_body, 0)
            sv = plsc.load_gather(pmat, [lanes, jnp.zeros((LANES,), jnp.int32)])
            for j in range(1, LANES):
                sv = sv + plsc.load_gather(
                    pmat, [lanes, jnp.full((LANES,), j, jnp.int32)])
            scores_v[pl.ds(t * CB, CB)] = sv

        fire(0, 0)

        def outer_body(k, carry):
            t = k * 2
            fire(t + 1, 1)
            drain(t, 0)
            compute(t, 0)

            @pl.when(t + 2 < t_chunks)
            def _():
                fire(t + 2, 0)

            drain(t + 1, 1)
            compute(t + 1, 1)
            return carry

        lax.fori_loop(0, t_chunks // 2, outer_body, 0)

        pltpu.sync_copy(scores_v, out_hbm.at[pl.ds(wid * ipw, ipw)])

    return sc_scores


def _loss_body(s_ref, o_ref):
    s = s_ref[...]
    half = s.shape[0] // 2
    pos = s[:half, :]
    neg = s[half:, :]
    tot = jnp.sum(jax.nn.log_sigmoid(pos)) + jnp.sum(jax.nn.log_sigmoid(-neg))
    o_ref[...] = jnp.reshape(-tot, (1, 1))


def kernel(pos_u, pos_v, neg_u, neg_v, u_table, v_table):
    b, ctx = pos_u.shape
    d = u_table.shape[1]
    n_items = 2 * b
    assert n_items % NW == 0
    ipw = n_items // NW
    assert ipw % CB == 0 and (ipw // CB) % 2 == 0
    assert (CB * ctx) % 4 == 0 and GATHER_ROWS == (CB * ctx) // 4
    assert GATHER_ROWS % ctx == 0

    uidx = jnp.concatenate(
        [pos_u.reshape(-1), neg_u.reshape(-1)]).astype(jnp.int32)
    vidx = jnp.concatenate([pos_v, neg_v]).astype(jnp.int32)

    scores = _make_sc_scores(n_items, ctx, d, ipw)(
        uidx, vidx, u_table, v_table)

    scores2d = scores.reshape(n_items // 128, 128)
    loss = pl.pallas_call(
        _loss_body,
        out_shape=jax.ShapeDtypeStruct((1, 1), jnp.float32),
    )(scores2d)
    return loss[0, 0]
